# baseline (speedup 1.0000x reference)
import jax
import jax.numpy as jnp
from jax import lax
from jax.experimental import pallas as pl
from jax.experimental.pallas import tpu as pltpu
from jax.experimental.pallas import tpu_sc as plsc

N = 10000
E = 160000
B = 128
DT = 128
DF = 256
H = 256
OUT = 256
NC = 10

NSUB = 16
LANES = 16
D2 = 128
NP_ = 10240
CK = 128
NCH = 80
EP = NSUB * NCH * CK
STRIPE = NP_ // NSUB
NST = 10
CPS = NCH // NST
NBUF = 2
S0 = 6

_MESH = dict(core_axis_name="c", subcore_axis_name="s")


def _common_scratch():
    return ([
        pltpu.VMEM((CPS, CK), jnp.int32),
        pltpu.VMEM((CPS, CK), jnp.int32),
    ] + [pltpu.VMEM((CK, D2), jnp.float32) for _ in range(NBUF)]
      + [pltpu.VMEM_SHARED((NP_, D2), jnp.float32)]
      + [pltpu.SemaphoreType.DMA for _ in range(NBUF)])


def _fill(ref, nrows, val):
    v16 = jnp.full((LANES,), val, jnp.float32)

    def row(i, c):
        for k in range(D2 // LANES):
            ref[i, pl.ds(k * LANES, LANES)] = v16
        return c
    lax.fori_loop(0, nrows, row, 0)


def _zero_acc(acc, zbuf, base):
    for k in range(STRIPE // CK):
        pltpu.sync_copy(zbuf, acc.at[pl.ds(base + k * CK, CK)])


def _scatter_pass(x_hbm, srcs_h, dsts_h, sid, src_v, dst_v, bufs, acc,
                  gsems, s_lo, s_hi):
    for s in range(s_lo, s_hi):
        pltpu.sync_copy(srcs_h.at[sid, pl.ds(s * CPS, CPS)], src_v)
        pltpu.sync_copy(dsts_h.at[sid, pl.ds(s * CPS, CPS)], dst_v)
        pltpu.async_copy(x_hbm.at[src_v.at[0]], bufs[0], gsems[0])

        def pair(p, c):
            j0 = 2 * p
            j1 = j0 + 1
            pltpu.make_async_copy(x_hbm.at[src_v.at[j0]], bufs[0],
                                  gsems[0]).wait()
            pltpu.async_copy(x_hbm.at[src_v.at[j1]], bufs[1], gsems[1])
            pltpu.sync_copy(bufs[0], acc.at[dst_v.at[j0]], add=True)
            pltpu.make_async_copy(x_hbm.at[src_v.at[j1]], bufs[1],
                                  gsems[1]).wait()

            @pl.when(p + 1 < CPS // 2)
            def _():
                pltpu.async_copy(x_hbm.at[src_v.at[j0 + 2]], bufs[0],
                                 gsems[0])
            pltpu.sync_copy(bufs[1], acc.at[dst_v.at[j1]], add=True)
            return c
        lax.fori_loop(0, CPS // 2, pair, 0)


def _ones_pass(ones, dsts_h, sid, dst_v, acc, s_lo, s_hi):
    for s in range(s_lo, s_hi):
        pltpu.sync_copy(dsts_h.at[sid, pl.ds(s * CPS, CPS)], dst_v)

        def chunk(j, c):
            pltpu.sync_copy(ones, acc.at[dst_v.at[j]], add=True)
            return c
        lax.fori_loop(0, CPS, chunk, 0)


def _segsum_pass1(topo_p, srcs, dsts):
    out_type = [jax.ShapeDtypeStruct((NP_, D2), jnp.float32)] * 3

    def body(x_h, srcs_h, dsts_h, out0, out1, dg, src_v, dst_v,
             b0, b1, acc, g0, g1):
        bufs = (b0, b1)
        gsems = (g0, g1)
        cid = lax.axis_index("c")
        sid = lax.axis_index("s")
        base = sid * STRIPE

        def stripe_out(dst_h):
            pltpu.sync_copy(acc.at[pl.ds(base, STRIPE)],
                            dst_h.at[pl.ds(base, STRIPE)])

        @pl.when(cid == 0)
        def _():
            _fill(b0, CK, 0.0)
            _zero_acc(acc, b0, base)
            plsc.subcore_barrier()
            _scatter_pass(x_h, srcs_h, dsts_h, sid, src_v, dst_v, bufs, acc,
                          gsems, 0, S0)
            plsc.subcore_barrier()
            stripe_out(out0)

        @pl.when(cid == 1)
        def _():
            _fill(b0, CK, 0.0)
            _fill(b1, CK, 1.0)
            _zero_acc(acc, b0, base)
            plsc.subcore_barrier()
            _ones_pass(b1, dsts_h, sid, dst_v, acc, 0, NST)
            plsc.subcore_barrier()
            stripe_out(dg)
            _zero_acc(acc, b0, base)
            plsc.subcore_barrier()
            _scatter_pass(x_h, srcs_h, dsts_h, sid, src_v, dst_v, bufs, acc,
                          gsems, S0, NST)
            plsc.subcore_barrier()
            stripe_out(out1)

    return pl.kernel(body, out_type=out_type,
                     mesh=plsc.VectorSubcoreMesh(**_MESH),
                     scratch_types=_common_scratch())(topo_p, srcs, dsts)


def _make_segsum(n_aggs):
    out_type = [jax.ShapeDtypeStruct((NP_, D2), jnp.float32)
                for _ in range(2 * n_aggs)]

    def body(*refs):
        xs = refs[:2 * n_aggs]
        srcs_h = refs[2 * n_aggs]
        dsts_h = refs[2 * n_aggs + 1]
        outs = refs[2 * n_aggs + 2:4 * n_aggs + 2]
        src_v, dst_v, b0, b1, acc, g0, g1 = refs[4 * n_aggs + 2:]
        bufs = (b0, b1)
        gsems = (g0, g1)

        cid = lax.axis_index("c")
        sid = lax.axis_index("s")
        base = sid * STRIPE

        for a in range(n_aggs):
            if a > 0:
                plsc.subcore_barrier()
            _fill(b0, CK, 0.0)
            _zero_acc(acc, b0, base)
            plsc.subcore_barrier()

            @pl.when(cid == 0)
            def _(a=a):
                _scatter_pass(xs[2 * a], srcs_h, dsts_h, sid, src_v, dst_v,
                              bufs, acc, gsems, 0, NST)

            @pl.when(cid == 1)
            def _(a=a):
                _scatter_pass(xs[2 * a + 1], srcs_h, dsts_h, sid, src_v,
                              dst_v, bufs, acc, gsems, 0, NST)

            plsc.subcore_barrier()

            @pl.when(cid == 0)
            def _(a=a):
                pltpu.sync_copy(acc.at[pl.ds(base, STRIPE)],
                                outs[2 * a].at[pl.ds(base, STRIPE)])

            @pl.when(cid == 1)
            def _(a=a):
                pltpu.sync_copy(acc.at[pl.ds(base, STRIPE)],
                                outs[2 * a + 1].at[pl.ds(base, STRIPE)])

    return pl.kernel(body, out_type=out_type,
                     mesh=plsc.VectorSubcoreMesh(**_MESH),
                     scratch_types=_common_scratch())


RB = 1024
NB = NP_ // RB

_row = lambda i: (i, 0)
_full = lambda i: (0, 0)


def _tc1(st0, st1, deg128, topo_p, feat_p, Ws1l, Ws1r, bs1):
    def body(ag0, ag1, dg, tp, ft, wl, wr, b1, h1L, h1R, y1L, y1R):
        deg0 = dg[:, 0:1]
        cnt = jnp.maximum(deg0, 1.0)
        mean = (ag0[...] + ag1[...]) / cnt
        h1 = jnp.maximum(
            jnp.dot(mean, wl[...], preferred_element_type=jnp.float32)
            + jnp.dot(tp[...], wr[...], preferred_element_type=jnp.float32)
            + b1[...], 0.0)
        dinv = lax.rsqrt(deg0 + 1.0)
        y1 = ft[...] * dinv
        h1L[...] = h1[:, :H // 2]
        h1R[...] = h1[:, H // 2:]
        y1L[...] = y1[:, :DF // 2]
        y1R[...] = y1[:, DF // 2:]

    return pl.pallas_call(
        body,
        grid=(NB,),
        in_specs=[
            pl.BlockSpec((RB, D2), _row),
            pl.BlockSpec((RB, D2), _row),
            pl.BlockSpec((RB, D2), _row),
            pl.BlockSpec((RB, DT), _row),
            pl.BlockSpec((RB, DF), _row),
            pl.BlockSpec((DT, H), _full),
            pl.BlockSpec((DT, H), _full),
            pl.BlockSpec((1, H), _full),
        ],
        out_specs=[pl.BlockSpec((RB, 128), _row)] * 4,
        out_shape=[jax.ShapeDtypeStruct((NP_, 128), jnp.float32)] * 4,
    )(st0, st1, deg128, topo_p, feat_p, Ws1l, Ws1r, bs1.reshape(1, H))


def _tc2(sh1L, sh1R, sy1L, sy1R, h1L, h1R, y1L, y1R, deg128, batch2d,
         Ws2l, Ws2r, bs2, Wg1, bg1):
    def body(shL, shR, syL, syR, hL, hR, yL, yR, dg, bt, wl, wr, b2, wg, bg,
             y2L, y2R, xtp):
        i = pl.program_id(0)
        deg0 = dg[:, 0:1]
        cnt = jnp.maximum(deg0, 1.0)
        dinv = lax.rsqrt(deg0 + 1.0)
        sh1 = jnp.concatenate([shL[...], shR[...]], axis=1)
        h1 = jnp.concatenate([hL[...], hR[...]], axis=1)
        xt = (jnp.dot(sh1 / cnt, wl[...], preferred_element_type=jnp.float32)
              + jnp.dot(h1, wr[...], preferred_element_type=jnp.float32)
              + b2[...])
        p1 = jnp.concatenate([syL[...] + yL[...], syR[...] + yR[...]], axis=1)
        g1 = jnp.maximum(
            jnp.dot(dinv * p1, wg[...], preferred_element_type=jnp.float32)
            + bg[...], 0.0)
        y2 = g1 * dinv
        y2L[...] = y2[:, :H // 2]
        y2R[...] = y2[:, H // 2:]
        oh = (bt[...] == lax.broadcasted_iota(jnp.int32, (RB, B), 1))
        contrib = lax.dot_general(oh.astype(jnp.float32), xt,
                                  (((0,), (0,)), ((), ())),
                                  preferred_element_type=jnp.float32)

        @pl.when(i == 0)
        def _():
            xtp[...] = jnp.zeros((B, OUT), jnp.float32)
        xtp[...] += contrib

    return pl.pallas_call(
        body,
        grid=(NB,),
        in_specs=[pl.BlockSpec((RB, 128), _row)] * 8 + [
            pl.BlockSpec((RB, D2), _row),
            pl.BlockSpec((RB, 1), _row),
            pl.BlockSpec((H, OUT), _full),
            pl.BlockSpec((H, OUT), _full),
            pl.BlockSpec((1, OUT), _full),
            pl.BlockSpec((DF, H), _full),
            pl.BlockSpec((1, H), _full),
        ],
        out_specs=[pl.BlockSpec((RB, 128), _row),
                   pl.BlockSpec((RB, 128), _row),
                   pl.BlockSpec((B, OUT), _full)],
        out_shape=[jax.ShapeDtypeStruct((NP_, 128), jnp.float32),
                   jax.ShapeDtypeStruct((NP_, 128), jnp.float32),
                   jax.ShapeDtypeStruct((B, OUT), jnp.float32)],
    )(sh1L, sh1R, sy1L, sy1R, h1L, h1R, y1L, y1R, deg128, batch2d,
      Ws2l, Ws2r, bs2.reshape(1, OUT), Wg1, bg1.reshape(1, H))


def _tc3(sy2L, sy2R, y2L, y2R, deg128, batch2d, Wg2, bg2, xt_pool,
         Wat, bat, Waf, baf, Wq, bq, Wl1, bl1, Wl2, bl2):
    _rowc = lambda i: (jnp.minimum(i, NB - 1), 0)

    def body(sL, sR, yL, yR, dg, bt, wg, bg, xtp, wat, bat_, waf, baf_,
             wq, bq_, wl1, bl1_, wl2, bl2_, out, xfp):
        i = pl.program_id(0)

        @pl.when(i == 0)
        def _():
            xfp[...] = jnp.zeros((B, OUT), jnp.float32)

        @pl.when(i < NB)
        def _():
            deg0 = dg[:, 0:1]
            dinv = lax.rsqrt(deg0 + 1.0)
            p2 = jnp.concatenate([sL[...] + yL[...], sR[...] + yR[...]],
                                 axis=1)
            xf = (jnp.dot(dinv * p2, wg[...],
                          preferred_element_type=jnp.float32) + bg[...])
            oh = (bt[...] == lax.broadcasted_iota(jnp.int32, (RB, B), 1))
            xfp[...] += lax.dot_general(oh.astype(jnp.float32), xf,
                                        (((0,), (0,)), ((), ())),
                                        preferred_element_type=jnp.float32)

        @pl.when(i == NB)
        def _():
            xt = xtp[...]
            xf = xfp[...]
            at = jnp.tanh(xt @ wat[...] + bat_[...]) @ wq[...] + bq_[...]
            af = jnp.tanh(xf @ waf[...] + baf_[...]) @ wq[...] + bq_[...]
            et = jnp.exp(at)
            ef = jnp.exp(af)
            x = (et * xt + ef * xf) / (et + ef)
            x = jnp.maximum(x @ wl1[...] + bl1_[...], 0.0)
            out[...] = x @ wl2[...] + bl2_[...]

    return pl.pallas_call(
        body,
        grid=(NB + 1,),
        in_specs=[pl.BlockSpec((RB, 128), _rowc)] * 4 + [
            pl.BlockSpec((RB, D2), _rowc),
            pl.BlockSpec((RB, 1), _rowc),
            pl.BlockSpec((H, OUT), _full),
            pl.BlockSpec((1, OUT), _full),
            pl.BlockSpec((B, OUT), _full),
            pl.BlockSpec((OUT, 32), _full),
            pl.BlockSpec((1, 32), _full),
            pl.BlockSpec((OUT, 32), _full),
            pl.BlockSpec((1, 32), _full),
            pl.BlockSpec((32, 1), _full),
            pl.BlockSpec((1, 1), _full),
            pl.BlockSpec((OUT, 32), _full),
            pl.BlockSpec((1, 32), _full),
            pl.BlockSpec((32, NC), _full),
            pl.BlockSpec((1, NC), _full),
        ],
        out_specs=pl.BlockSpec((B, NC), _full),
        out_shape=jax.ShapeDtypeStruct((B, NC), jnp.float32),
        scratch_shapes=[pltpu.VMEM((B, OUT), jnp.float32)],
    )(sy2L, sy2R, y2L, y2R, deg128, batch2d, Wg2, bg2.reshape(1, OUT),
      xt_pool, Wat, bat.reshape(1, 32), Waf, baf.reshape(1, 32), Wq,
      bq.reshape(1, 1), Wl1, bl1.reshape(1, 32), Wl2, bl2.reshape(1, NC))


def kernel(feat, topo, edge_index, batch, Ws1l, Ws1r, bs1, Ws2l, Ws2r, bs2,
           Wg1, bg1, Wg2, bg2, Wat, bat, Waf, baf, Wq, bq, Wl1, bl1, Wl2, bl2):
    epad = jnp.pad(edge_index, ((0, 0), (0, EP - E)), constant_values=N)
    srcs = epad[0].reshape(NSUB, NCH, CK)
    dsts = epad[1].reshape(NSUB, NCH, CK)
    topo_p = jnp.pad(topo, ((0, NP_ - N), (0, 0)))
    feat_p = jnp.pad(feat, ((0, NP_ - N), (0, 0)))
    batch2d = jnp.pad(batch, (0, NP_ - N), constant_values=B).reshape(NP_, 1)

    st0, st1, deg128 = _segsum_pass1(topo_p, srcs, dsts)
    h1L, h1R, y1L, y1R = _tc1(st0, st1, deg128, topo_p, feat_p,
                              Ws1l, Ws1r, bs1)
    sh1L, sh1R, sy1L, sy1R = _make_segsum(2)(h1L, h1R, y1L, y1R, srcs, dsts)
    y2L, y2R, xt_pool = _tc2(sh1L, sh1R, sy1L, sy1R, h1L, h1R, y1L, y1R,
                             deg128, batch2d, Ws2l, Ws2r, bs2, Wg1, bg1)
    sy2L, sy2R = _make_segsum(1)(y2L, y2R, srcs, dsts)
    return _tc3(sy2L, sy2R, y2L, y2R, deg128, batch2d, Wg2, bg2, xt_pool,
                Wat, bat, Waf, baf, Wq, bq, Wl1, bl1, Wl2, bl2)


# speedup vs baseline: 1.0318x; 1.0318x over previous
import jax
import jax.numpy as jnp
from jax import lax
from jax.experimental import pallas as pl
from jax.experimental.pallas import tpu as pltpu
from jax.experimental.pallas import tpu_sc as plsc

N = 10000
E = 160000
B = 128
DT = 128
DF = 256
H = 256
OUT = 256
NC = 10

NSUB = 16
LANES = 16
D2 = 128
NP_ = 10240
CK = 128
NCH = 80
EP = NSUB * NCH * CK
STRIPE = NP_ // NSUB
NBUF = 2
CPS1 = 8
NST1 = NCH // CPS1
S0 = 6
CPS2 = 40
NST2 = NCH // CPS2

_MESH = dict(core_axis_name="c", subcore_axis_name="s")


def _common_scratch(cps):
    return ([
        pltpu.VMEM((cps, CK), jnp.int32),
        pltpu.VMEM((cps, CK), jnp.int32),
    ] + [pltpu.VMEM((CK, D2), jnp.float32) for _ in range(NBUF)]
      + [pltpu.VMEM_SHARED((NP_, D2), jnp.float32)]
      + [pltpu.SemaphoreType.DMA for _ in range(NBUF)])


def _fill(ref, nrows, val):
    v16 = jnp.full((LANES,), val, jnp.float32)

    def row(i, c):
        for k in range(D2 // LANES):
            ref[i, pl.ds(k * LANES, LANES)] = v16
        return c
    lax.fori_loop(0, nrows, row, 0)


def _zero_acc(acc, zbuf, base):
    for k in range(STRIPE // CK):
        pltpu.sync_copy(zbuf, acc.at[pl.ds(base + k * CK, CK)])


def _scatter_pass(x_hbm, srcs_h, dsts_h, sid, src_v, dst_v, bufs, acc,
                  gsems, cps, s_lo, s_hi):
    for s in range(s_lo, s_hi):
        pltpu.sync_copy(srcs_h.at[sid, pl.ds(s * cps, cps)], src_v)
        pltpu.sync_copy(dsts_h.at[sid, pl.ds(s * cps, cps)], dst_v)
        pltpu.async_copy(x_hbm.at[src_v.at[0]], bufs[0], gsems[0])

        def pair(p, c):
            j0 = 2 * p
            j1 = j0 + 1
            pltpu.make_async_copy(x_hbm.at[src_v.at[j0]], bufs[0],
                                  gsems[0]).wait()
            pltpu.async_copy(x_hbm.at[src_v.at[j1]], bufs[1], gsems[1])
            pltpu.sync_copy(bufs[0], acc.at[dst_v.at[j0]], add=True)
            pltpu.make_async_copy(x_hbm.at[src_v.at[j1]], bufs[1],
                                  gsems[1]).wait()

            @pl.when(p + 1 < cps // 2)
            def _():
                pltpu.async_copy(x_hbm.at[src_v.at[j0 + 2]], bufs[0],
                                 gsems[0])
            pltpu.sync_copy(bufs[1], acc.at[dst_v.at[j1]], add=True)
            return c
        lax.fori_loop(0, cps // 2, pair, 0)


def _ones_pass(ones, dsts_h, sid, dst_v, acc, cps, s_lo, s_hi):
    for s in range(s_lo, s_hi):
        pltpu.sync_copy(dsts_h.at[sid, pl.ds(s * cps, cps)], dst_v)

        def chunk(j, c):
            pltpu.sync_copy(ones, acc.at[dst_v.at[j]], add=True)
            return c
        lax.fori_loop(0, cps, chunk, 0)


def _segsum_pass1(topo_p, srcs, dsts):
    out_type = [jax.ShapeDtypeStruct((NP_, D2), jnp.float32)] * 3

    def body(x_h, srcs_h, dsts_h, out0, out1, dg, src_v, dst_v,
             b0, b1, acc, g0, g1):
        bufs = (b0, b1)
        gsems = (g0, g1)
        cid = lax.axis_index("c")
        sid = lax.axis_index("s")
        base = sid * STRIPE

        def stripe_out(dst_h):
            pltpu.sync_copy(acc.at[pl.ds(base, STRIPE)],
                            dst_h.at[pl.ds(base, STRIPE)])

        @pl.when(cid == 0)
        def _():
            _fill(b0, CK, 0.0)
            _zero_acc(acc, b0, base)
            plsc.subcore_barrier()
            _scatter_pass(x_h, srcs_h, dsts_h, sid, src_v, dst_v, bufs, acc,
                          gsems, CPS1, 0, S0)
            plsc.subcore_barrier()
            stripe_out(out0)

        @pl.when(cid == 1)
        def _():
            _fill(b0, CK, 0.0)
            _fill(b1, CK, 1.0)
            _zero_acc(acc, b0, base)
            plsc.subcore_barrier()
            _ones_pass(b1, dsts_h, sid, dst_v, acc, CPS1, 0, NST1)
            plsc.subcore_barrier()
            stripe_out(dg)
            _zero_acc(acc, b0, base)
            plsc.subcore_barrier()
            _scatter_pass(x_h, srcs_h, dsts_h, sid, src_v, dst_v, bufs, acc,
                          gsems, CPS1, S0, NST1)
            plsc.subcore_barrier()
            stripe_out(out1)

    return pl.kernel(body, out_type=out_type,
                     mesh=plsc.VectorSubcoreMesh(**_MESH),
                     scratch_types=_common_scratch(CPS1))(topo_p, srcs, dsts)


def _make_segsum(n_aggs):
    out_type = [jax.ShapeDtypeStruct((NP_, D2), jnp.float32)
                for _ in range(2 * n_aggs)]

    def body(*refs):
        xs = refs[:2 * n_aggs]
        srcs_h = refs[2 * n_aggs]
        dsts_h = refs[2 * n_aggs + 1]
        outs = refs[2 * n_aggs + 2:4 * n_aggs + 2]
        src_v, dst_v, b0, b1, acc, g0, g1 = refs[4 * n_aggs + 2:]
        bufs = (b0, b1)
        gsems = (g0, g1)

        cid = lax.axis_index("c")
        sid = lax.axis_index("s")
        base = sid * STRIPE

        for a in range(n_aggs):
            if a > 0:
                plsc.subcore_barrier()
            _fill(b0, CK, 0.0)
            _zero_acc(acc, b0, base)
            plsc.subcore_barrier()

            @pl.when(cid == 0)
            def _(a=a):
                _scatter_pass(xs[2 * a], srcs_h, dsts_h, sid, src_v, dst_v,
                              bufs, acc, gsems, CPS2, 0, NST2)

            @pl.when(cid == 1)
            def _(a=a):
                _scatter_pass(xs[2 * a + 1], srcs_h, dsts_h, sid, src_v,
                              dst_v, bufs, acc, gsems, CPS2, 0, NST2)

            plsc.subcore_barrier()

            @pl.when(cid == 0)
            def _(a=a):
                pltpu.sync_copy(acc.at[pl.ds(base, STRIPE)],
                                outs[2 * a].at[pl.ds(base, STRIPE)])

            @pl.when(cid == 1)
            def _(a=a):
                pltpu.sync_copy(acc.at[pl.ds(base, STRIPE)],
                                outs[2 * a + 1].at[pl.ds(base, STRIPE)])

    return pl.kernel(body, out_type=out_type,
                     mesh=plsc.VectorSubcoreMesh(**_MESH),
                     scratch_types=_common_scratch(CPS2))


RB = 1024
NB = NP_ // RB

_row = lambda i: (i, 0)
_full = lambda i: (0, 0)


def _tc1(st0, st1, deg128, topo_p, feat_p, Ws1l, Ws1r, bs1):
    def body(ag0, ag1, dg, tp, ft, wl, wr, b1, h1L, h1R, y1L, y1R):
        deg0 = dg[:, 0:1]
        cnt = jnp.maximum(deg0, 1.0)
        mean = (ag0[...] + ag1[...]) / cnt
        h1 = jnp.maximum(
            jnp.dot(mean, wl[...], preferred_element_type=jnp.float32)
            + jnp.dot(tp[...], wr[...], preferred_element_type=jnp.float32)
            + b1[...], 0.0)
        dinv = lax.rsqrt(deg0 + 1.0)
        y1 = ft[...] * dinv
        h1L[...] = h1[:, :H // 2]
        h1R[...] = h1[:, H // 2:]
        y1L[...] = y1[:, :DF // 2]
        y1R[...] = y1[:, DF // 2:]

    return pl.pallas_call(
        body,
        grid=(NB,),
        in_specs=[
            pl.BlockSpec((RB, D2), _row),
            pl.BlockSpec((RB, D2), _row),
            pl.BlockSpec((RB, D2), _row),
            pl.BlockSpec((RB, DT), _row),
            pl.BlockSpec((RB, DF), _row),
            pl.BlockSpec((DT, H), _full),
            pl.BlockSpec((DT, H), _full),
            pl.BlockSpec((1, H), _full),
        ],
        out_specs=[pl.BlockSpec((RB, 128), _row)] * 4,
        out_shape=[jax.ShapeDtypeStruct((NP_, 128), jnp.float32)] * 4,
    )(st0, st1, deg128, topo_p, feat_p, Ws1l, Ws1r, bs1.reshape(1, H))


def _tc2(sh1L, sh1R, sy1L, sy1R, h1L, h1R, y1L, y1R, deg128, batch2d,
         Ws2l, Ws2r, bs2, Wg1, bg1):
    def body(shL, shR, syL, syR, hL, hR, yL, yR, dg, bt, wl, wr, b2, wg, bg,
             y2L, y2R, xtp):
        i = pl.program_id(0)
        deg0 = dg[:, 0:1]
        cnt = jnp.maximum(deg0, 1.0)
        dinv = lax.rsqrt(deg0 + 1.0)
        sh1 = jnp.concatenate([shL[...], shR[...]], axis=1)
        h1 = jnp.concatenate([hL[...], hR[...]], axis=1)
        xt = (jnp.dot(sh1 / cnt, wl[...], preferred_element_type=jnp.float32)
              + jnp.dot(h1, wr[...], preferred_element_type=jnp.float32)
              + b2[...])
        p1 = jnp.concatenate([syL[...] + yL[...], syR[...] + yR[...]], axis=1)
        g1 = jnp.maximum(
            jnp.dot(dinv * p1, wg[...], preferred_element_type=jnp.float32)
            + bg[...], 0.0)
        y2 = g1 * dinv
        y2L[...] = y2[:, :H // 2]
        y2R[...] = y2[:, H // 2:]
        oh = (bt[...] == lax.broadcasted_iota(jnp.int32, (RB, B), 1))
        contrib = lax.dot_general(oh.astype(jnp.float32), xt,
                                  (((0,), (0,)), ((), ())),
                                  preferred_element_type=jnp.float32)

        @pl.when(i == 0)
        def _():
            xtp[...] = jnp.zeros((B, OUT), jnp.float32)
        xtp[...] += contrib

    return pl.pallas_call(
        body,
        grid=(NB,),
        in_specs=[pl.BlockSpec((RB, 128), _row)] * 8 + [
            pl.BlockSpec((RB, D2), _row),
            pl.BlockSpec((RB, 1), _row),
            pl.BlockSpec((H, OUT), _full),
            pl.BlockSpec((H, OUT), _full),
            pl.BlockSpec((1, OUT), _full),
            pl.BlockSpec((DF, H), _full),
            pl.BlockSpec((1, H), _full),
        ],
        out_specs=[pl.BlockSpec((RB, 128), _row),
                   pl.BlockSpec((RB, 128), _row),
                   pl.BlockSpec((B, OUT), _full)],
        out_shape=[jax.ShapeDtypeStruct((NP_, 128), jnp.float32),
                   jax.ShapeDtypeStruct((NP_, 128), jnp.float32),
                   jax.ShapeDtypeStruct((B, OUT), jnp.float32)],
    )(sh1L, sh1R, sy1L, sy1R, h1L, h1R, y1L, y1R, deg128, batch2d,
      Ws2l, Ws2r, bs2.reshape(1, OUT), Wg1, bg1.reshape(1, H))


def _tc3(sy2L, sy2R, y2L, y2R, deg128, batch2d, Wg2, bg2, xt_pool,
         Wat, bat, Waf, baf, Wq, bq, Wl1, bl1, Wl2, bl2):
    _rowc = lambda i: (jnp.minimum(i, NB - 1), 0)

    def body(sL, sR, yL, yR, dg, bt, wg, bg, xtp, wat, bat_, waf, baf_,
             wq, bq_, wl1, bl1_, wl2, bl2_, out, xfp):
        i = pl.program_id(0)

        @pl.when(i == 0)
        def _():
            xfp[...] = jnp.zeros((B, OUT), jnp.float32)

        @pl.when(i < NB)
        def _():
            deg0 = dg[:, 0:1]
            dinv = lax.rsqrt(deg0 + 1.0)
            p2 = jnp.concatenate([sL[...] + yL[...], sR[...] + yR[...]],
                                 axis=1)
            xf = (jnp.dot(dinv * p2, wg[...],
                          preferred_element_type=jnp.float32) + bg[...])
            oh = (bt[...] == lax.broadcasted_iota(jnp.int32, (RB, B), 1))
            xfp[...] += lax.dot_general(oh.astype(jnp.float32), xf,
                                        (((0,), (0,)), ((), ())),
                                        preferred_element_type=jnp.float32)

        @pl.when(i == NB)
        def _():
            xt = xtp[...]
            xf = xfp[...]
            at = jnp.tanh(xt @ wat[...] + bat_[...]) @ wq[...] + bq_[...]
            af = jnp.tanh(xf @ waf[...] + baf_[...]) @ wq[...] + bq_[...]
            et = jnp.exp(at)
            ef = jnp.exp(af)
            x = (et * xt + ef * xf) / (et + ef)
            x = jnp.maximum(x @ wl1[...] + bl1_[...], 0.0)
            out[...] = x @ wl2[...] + bl2_[...]

    return pl.pallas_call(
        body,
        grid=(NB + 1,),
        in_specs=[pl.BlockSpec((RB, 128), _rowc)] * 4 + [
            pl.BlockSpec((RB, D2), _rowc),
            pl.BlockSpec((RB, 1), _rowc),
            pl.BlockSpec((H, OUT), _full),
            pl.BlockSpec((1, OUT), _full),
            pl.BlockSpec((B, OUT), _full),
            pl.BlockSpec((OUT, 32), _full),
            pl.BlockSpec((1, 32), _full),
            pl.BlockSpec((OUT, 32), _full),
            pl.BlockSpec((1, 32), _full),
            pl.BlockSpec((32, 1), _full),
            pl.BlockSpec((1, 1), _full),
            pl.BlockSpec((OUT, 32), _full),
            pl.BlockSpec((1, 32), _full),
            pl.BlockSpec((32, NC), _full),
            pl.BlockSpec((1, NC), _full),
        ],
        out_specs=pl.BlockSpec((B, NC), _full),
        out_shape=jax.ShapeDtypeStruct((B, NC), jnp.float32),
        scratch_shapes=[pltpu.VMEM((B, OUT), jnp.float32)],
    )(sy2L, sy2R, y2L, y2R, deg128, batch2d, Wg2, bg2.reshape(1, OUT),
      xt_pool, Wat, bat.reshape(1, 32), Waf, baf.reshape(1, 32), Wq,
      bq.reshape(1, 1), Wl1, bl1.reshape(1, 32), Wl2, bl2.reshape(1, NC))


def kernel(feat, topo, edge_index, batch, Ws1l, Ws1r, bs1, Ws2l, Ws2r, bs2,
           Wg1, bg1, Wg2, bg2, Wat, bat, Waf, baf, Wq, bq, Wl1, bl1, Wl2, bl2):
    epad = jnp.pad(edge_index, ((0, 0), (0, EP - E)), constant_values=N)
    srcs = epad[0].reshape(NSUB, NCH, CK)
    dsts = epad[1].reshape(NSUB, NCH, CK)
    topo_p = jnp.pad(topo, ((0, NP_ - N), (0, 0)))
    feat_p = jnp.pad(feat, ((0, NP_ - N), (0, 0)))
    batch2d = jnp.pad(batch, (0, NP_ - N), constant_values=B).reshape(NP_, 1)

    st0, st1, deg128 = _segsum_pass1(topo_p, srcs, dsts)
    h1L, h1R, y1L, y1R = _tc1(st0, st1, deg128, topo_p, feat_p,
                              Ws1l, Ws1r, bs1)
    sh1L, sh1R, sy1L, sy1R = _make_segsum(2)(h1L, h1R, y1L, y1R, srcs, dsts)
    y2L, y2R, xt_pool = _tc2(sh1L, sh1R, sy1L, sy1R, h1L, h1R, y1L, y1R,
                             deg128, batch2d, Ws2l, Ws2r, bs2, Wg1, bg1)
    sy2L, sy2R = _make_segsum(1)(y2L, y2R, srcs, dsts)
    return _tc3(sy2L, sy2R, y2L, y2R, deg128, batch2d, Wg2, bg2, xt_pool,
                Wat, bat, Waf, baf, Wq, bq, Wl1, bl1, Wl2, bl2)


# speedup vs baseline: 1.0571x; 1.0245x over previous
import jax
import jax.numpy as jnp
from jax import lax
from jax.experimental import pallas as pl
from jax.experimental.pallas import tpu as pltpu
from jax.experimental.pallas import tpu_sc as plsc

N = 10000
E = 160000
B = 128
DT = 128
DF = 256
H = 256
OUT = 256
NC = 10

NSUB = 16
LANES = 16
D2 = 128
NP_ = 10240
CK = 128
NCH = 80
EP = NSUB * NCH * CK
STRIPE = NP_ // NSUB
NBUF = 2
CPS1 = 40
NST1 = NCH // CPS1
S0 = NST1
CPS2 = 40
NST2 = NCH // CPS2

_MESH = dict(core_axis_name="c", subcore_axis_name="s")


def _common_scratch(cps):
    return ([
        pltpu.VMEM((cps, CK), jnp.int32),
        pltpu.VMEM((cps, CK), jnp.int32),
    ] + [pltpu.VMEM((CK, D2), jnp.float32) for _ in range(NBUF)]
      + [pltpu.VMEM_SHARED((NP_, D2), jnp.float32)]
      + [pltpu.SemaphoreType.DMA for _ in range(NBUF)])


def _fill(ref, nrows, val):
    v16 = jnp.full((LANES,), val, jnp.float32)

    def row(i, c):
        for k in range(D2 // LANES):
            ref[i, pl.ds(k * LANES, LANES)] = v16
        return c
    lax.fori_loop(0, nrows, row, 0)


def _zero_acc(acc, zbuf, base):
    for k in range(STRIPE // CK):
        pltpu.sync_copy(zbuf, acc.at[pl.ds(base + k * CK, CK)])


def _scatter_pass(x_hbm, srcs_h, dsts_h, sid, src_v, dst_v, bufs, acc,
                  gsems, cps, s_lo, s_hi):
    for s in range(s_lo, s_hi):
        pltpu.sync_copy(srcs_h.at[sid, pl.ds(s * cps, cps)], src_v)
        pltpu.sync_copy(dsts_h.at[sid, pl.ds(s * cps, cps)], dst_v)
        pltpu.async_copy(x_hbm.at[src_v.at[0]], bufs[0], gsems[0])

        def pair(p, c):
            j0 = 2 * p
            j1 = j0 + 1
            pltpu.make_async_copy(x_hbm.at[src_v.at[j0]], bufs[0],
                                  gsems[0]).wait()
            pltpu.async_copy(x_hbm.at[src_v.at[j1]], bufs[1], gsems[1])
            pltpu.sync_copy(bufs[0], acc.at[dst_v.at[j0]], add=True)
            pltpu.make_async_copy(x_hbm.at[src_v.at[j1]], bufs[1],
                                  gsems[1]).wait()

            @pl.when(p + 1 < cps // 2)
            def _():
                pltpu.async_copy(x_hbm.at[src_v.at[j0 + 2]], bufs[0],
                                 gsems[0])
            pltpu.sync_copy(bufs[1], acc.at[dst_v.at[j1]], add=True)
            return c
        lax.fori_loop(0, cps // 2, pair, 0)


def _ones_pass(ones, dsts_h, sid, dst_v, acc, cps, s_lo, s_hi):
    for s in range(s_lo, s_hi):
        pltpu.sync_copy(dsts_h.at[sid, pl.ds(s * cps, cps)], dst_v)

        def chunk(j, c):
            pltpu.sync_copy(ones, acc.at[dst_v.at[j]], add=True)
            return c
        lax.fori_loop(0, cps, chunk, 0)


def _segsum_pass1(topo_p, srcs, dsts):
    out_type = [jax.ShapeDtypeStruct((NP_, D2), jnp.float32)] * 3

    def body(x_h, srcs_h, dsts_h, out0, out1, dg, src_v, dst_v,
             b0, b1, acc, g0, g1):
        bufs = (b0, b1)
        gsems = (g0, g1)
        cid = lax.axis_index("c")
        sid = lax.axis_index("s")
        base = sid * STRIPE

        def stripe_out(dst_h):
            pltpu.sync_copy(acc.at[pl.ds(base, STRIPE)],
                            dst_h.at[pl.ds(base, STRIPE)])

        @pl.when(cid == 0)
        def _():
            _fill(b0, CK, 0.0)
            _zero_acc(acc, b0, base)
            plsc.subcore_barrier()
            _scatter_pass(x_h, srcs_h, dsts_h, sid, src_v, dst_v, bufs, acc,
                          gsems, CPS1, 0, S0)
            plsc.subcore_barrier()
            stripe_out(out0)

        @pl.when(cid == 1)
        def _():
            _fill(b0, CK, 0.0)
            _fill(b1, CK, 1.0)
            _zero_acc(acc, b0, base)
            plsc.subcore_barrier()
            _ones_pass(b1, dsts_h, sid, dst_v, acc, CPS1, 0, NST1)
            plsc.subcore_barrier()
            stripe_out(dg)
            _zero_acc(acc, b0, base)
            plsc.subcore_barrier()
            _scatter_pass(x_h, srcs_h, dsts_h, sid, src_v, dst_v, bufs, acc,
                          gsems, CPS1, S0, NST1)
            plsc.subcore_barrier()
            stripe_out(out1)

    return pl.kernel(body, out_type=out_type,
                     mesh=plsc.VectorSubcoreMesh(**_MESH),
                     scratch_types=_common_scratch(CPS1))(topo_p, srcs, dsts)


def _make_segsum(n_aggs):
    out_type = [jax.ShapeDtypeStruct((NP_, D2), jnp.float32)
                for _ in range(2 * n_aggs)]

    def body(*refs):
        xs = refs[:2 * n_aggs]
        srcs_h = refs[2 * n_aggs]
        dsts_h = refs[2 * n_aggs + 1]
        outs = refs[2 * n_aggs + 2:4 * n_aggs + 2]
        src_v, dst_v, b0, b1, acc, g0, g1 = refs[4 * n_aggs + 2:]
        bufs = (b0, b1)
        gsems = (g0, g1)

        cid = lax.axis_index("c")
        sid = lax.axis_index("s")
        base = sid * STRIPE

        for a in range(n_aggs):
            if a > 0:
                plsc.subcore_barrier()
            _fill(b0, CK, 0.0)
            _zero_acc(acc, b0, base)
            plsc.subcore_barrier()

            @pl.when(cid == 0)
            def _(a=a):
                _scatter_pass(xs[2 * a], srcs_h, dsts_h, sid, src_v, dst_v,
                              bufs, acc, gsems, CPS2, 0, NST2)

            @pl.when(cid == 1)
            def _(a=a):
                _scatter_pass(xs[2 * a + 1], srcs_h, dsts_h, sid, src_v,
                              dst_v, bufs, acc, gsems, CPS2, 0, NST2)

            plsc.subcore_barrier()

            @pl.when(cid == 0)
            def _(a=a):
                pltpu.sync_copy(acc.at[pl.ds(base, STRIPE)],
                                outs[2 * a].at[pl.ds(base, STRIPE)])

            @pl.when(cid == 1)
            def _(a=a):
                pltpu.sync_copy(acc.at[pl.ds(base, STRIPE)],
                                outs[2 * a + 1].at[pl.ds(base, STRIPE)])

    return pl.kernel(body, out_type=out_type,
                     mesh=plsc.VectorSubcoreMesh(**_MESH),
                     scratch_types=_common_scratch(CPS2))


RB = 1024
NB = NP_ // RB

_row = lambda i: (i, 0)
_full = lambda i: (0, 0)


def _tc1(st0, st1, deg128, topo_p, feat_p, Ws1l, Ws1r, bs1):
    def body(ag0, ag1, dg, tp, ft, wl, wr, b1, h1L, h1R, y1L, y1R):
        deg0 = dg[:, 0:1]
        cnt = jnp.maximum(deg0, 1.0)
        mean = (ag0[...] + ag1[...]) / cnt
        h1 = jnp.maximum(
            jnp.dot(mean, wl[...], preferred_element_type=jnp.float32)
            + jnp.dot(tp[...], wr[...], preferred_element_type=jnp.float32)
            + b1[...], 0.0)
        dinv = lax.rsqrt(deg0 + 1.0)
        y1 = ft[...] * dinv
        h1L[...] = h1[:, :H // 2]
        h1R[...] = h1[:, H // 2:]
        y1L[...] = y1[:, :DF // 2]
        y1R[...] = y1[:, DF // 2:]

    return pl.pallas_call(
        body,
        grid=(NB,),
        in_specs=[
            pl.BlockSpec((RB, D2), _row),
            pl.BlockSpec((RB, D2), _row),
            pl.BlockSpec((RB, D2), _row),
            pl.BlockSpec((RB, DT), _row),
            pl.BlockSpec((RB, DF), _row),
            pl.BlockSpec((DT, H), _full),
            pl.BlockSpec((DT, H), _full),
            pl.BlockSpec((1, H), _full),
        ],
        out_specs=[pl.BlockSpec((RB, 128), _row)] * 4,
        out_shape=[jax.ShapeDtypeStruct((NP_, 128), jnp.float32)] * 4,
    )(st0, st1, deg128, topo_p, feat_p, Ws1l, Ws1r, bs1.reshape(1, H))


def _tc2(sh1L, sh1R, sy1L, sy1R, h1L, h1R, y1L, y1R, deg128, batch2d,
         Ws2l, Ws2r, bs2, Wg1, bg1):
    def body(shL, shR, syL, syR, hL, hR, yL, yR, dg, bt, wl, wr, b2, wg, bg,
             y2L, y2R, xtp):
        i = pl.program_id(0)
        deg0 = dg[:, 0:1]
        cnt = jnp.maximum(deg0, 1.0)
        dinv = lax.rsqrt(deg0 + 1.0)
        sh1 = jnp.concatenate([shL[...], shR[...]], axis=1)
        h1 = jnp.concatenate([hL[...], hR[...]], axis=1)
        xt = (jnp.dot(sh1 / cnt, wl[...], preferred_element_type=jnp.float32)
              + jnp.dot(h1, wr[...], preferred_element_type=jnp.float32)
              + b2[...])
        p1 = jnp.concatenate([syL[...] + yL[...], syR[...] + yR[...]], axis=1)
        g1 = jnp.maximum(
            jnp.dot(dinv * p1, wg[...], preferred_element_type=jnp.float32)
            + bg[...], 0.0)
        y2 = g1 * dinv
        y2L[...] = y2[:, :H // 2]
        y2R[...] = y2[:, H // 2:]
        oh = (bt[...] == lax.broadcasted_iota(jnp.int32, (RB, B), 1))
        contrib = lax.dot_general(oh.astype(jnp.float32), xt,
                                  (((0,), (0,)), ((), ())),
                                  preferred_element_type=jnp.float32)

        @pl.when(i == 0)
        def _():
            xtp[...] = jnp.zeros((B, OUT), jnp.float32)
        xtp[...] += contrib

    return pl.pallas_call(
        body,
        grid=(NB,),
        in_specs=[pl.BlockSpec((RB, 128), _row)] * 8 + [
            pl.BlockSpec((RB, D2), _row),
            pl.BlockSpec((RB, 1), _row),
            pl.BlockSpec((H, OUT), _full),
            pl.BlockSpec((H, OUT), _full),
            pl.BlockSpec((1, OUT), _full),
            pl.BlockSpec((DF, H), _full),
            pl.BlockSpec((1, H), _full),
        ],
        out_specs=[pl.BlockSpec((RB, 128), _row),
                   pl.BlockSpec((RB, 128), _row),
                   pl.BlockSpec((B, OUT), _full)],
        out_shape=[jax.ShapeDtypeStruct((NP_, 128), jnp.float32),
                   jax.ShapeDtypeStruct((NP_, 128), jnp.float32),
                   jax.ShapeDtypeStruct((B, OUT), jnp.float32)],
    )(sh1L, sh1R, sy1L, sy1R, h1L, h1R, y1L, y1R, deg128, batch2d,
      Ws2l, Ws2r, bs2.reshape(1, OUT), Wg1, bg1.reshape(1, H))


def _tc3(sy2L, sy2R, y2L, y2R, deg128, batch2d, Wg2, bg2, xt_pool,
         Wat, bat, Waf, baf, Wq, bq, Wl1, bl1, Wl2, bl2):
    _rowc = lambda i: (jnp.minimum(i, NB - 1), 0)

    def body(sL, sR, yL, yR, dg, bt, wg, bg, xtp, wat, bat_, waf, baf_,
             wq, bq_, wl1, bl1_, wl2, bl2_, out, xfp):
        i = pl.program_id(0)

        @pl.when(i == 0)
        def _():
            xfp[...] = jnp.zeros((B, OUT), jnp.float32)

        @pl.when(i < NB)
        def _():
            deg0 = dg[:, 0:1]
            dinv = lax.rsqrt(deg0 + 1.0)
            p2 = jnp.concatenate([sL[...] + yL[...], sR[...] + yR[...]],
                                 axis=1)
            xf = (jnp.dot(dinv * p2, wg[...],
                          preferred_element_type=jnp.float32) + bg[...])
            oh = (bt[...] == lax.broadcasted_iota(jnp.int32, (RB, B), 1))
            xfp[...] += lax.dot_general(oh.astype(jnp.float32), xf,
                                        (((0,), (0,)), ((), ())),
                                        preferred_element_type=jnp.float32)

        @pl.when(i == NB)
        def _():
            xt = xtp[...]
            xf = xfp[...]
            at = jnp.tanh(xt @ wat[...] + bat_[...]) @ wq[...] + bq_[...]
            af = jnp.tanh(xf @ waf[...] + baf_[...]) @ wq[...] + bq_[...]
            et = jnp.exp(at)
            ef = jnp.exp(af)
            x = (et * xt + ef * xf) / (et + ef)
            x = jnp.maximum(x @ wl1[...] + bl1_[...], 0.0)
            out[...] = x @ wl2[...] + bl2_[...]

    return pl.pallas_call(
        body,
        grid=(NB + 1,),
        in_specs=[pl.BlockSpec((RB, 128), _rowc)] * 4 + [
            pl.BlockSpec((RB, D2), _rowc),
            pl.BlockSpec((RB, 1), _rowc),
            pl.BlockSpec((H, OUT), _full),
            pl.BlockSpec((1, OUT), _full),
            pl.BlockSpec((B, OUT), _full),
            pl.BlockSpec((OUT, 32), _full),
            pl.BlockSpec((1, 32), _full),
            pl.BlockSpec((OUT, 32), _full),
            pl.BlockSpec((1, 32), _full),
            pl.BlockSpec((32, 1), _full),
            pl.BlockSpec((1, 1), _full),
            pl.BlockSpec((OUT, 32), _full),
            pl.BlockSpec((1, 32), _full),
            pl.BlockSpec((32, NC), _full),
            pl.BlockSpec((1, NC), _full),
        ],
        out_specs=pl.BlockSpec((B, NC), _full),
        out_shape=jax.ShapeDtypeStruct((B, NC), jnp.float32),
        scratch_shapes=[pltpu.VMEM((B, OUT), jnp.float32)],
    )(sy2L, sy2R, y2L, y2R, deg128, batch2d, Wg2, bg2.reshape(1, OUT),
      xt_pool, Wat, bat.reshape(1, 32), Waf, baf.reshape(1, 32), Wq,
      bq.reshape(1, 1), Wl1, bl1.reshape(1, 32), Wl2, bl2.reshape(1, NC))


def kernel(feat, topo, edge_index, batch, Ws1l, Ws1r, bs1, Ws2l, Ws2r, bs2,
           Wg1, bg1, Wg2, bg2, Wat, bat, Waf, baf, Wq, bq, Wl1, bl1, Wl2, bl2):
    epad = jnp.pad(edge_index, ((0, 0), (0, EP - E)), constant_values=N)
    srcs = epad[0].reshape(NSUB, NCH, CK)
    dsts = epad[1].reshape(NSUB, NCH, CK)
    topo_p = jnp.pad(topo, ((0, NP_ - N), (0, 0)))
    feat_p = jnp.pad(feat, ((0, NP_ - N), (0, 0)))
    batch2d = jnp.pad(batch, (0, NP_ - N), constant_values=B).reshape(NP_, 1)

    st0, st1, deg128 = _segsum_pass1(topo_p, srcs, dsts)
    h1L, h1R, y1L, y1R = _tc1(st0, st1, deg128, topo_p, feat_p,
                              Ws1l, Ws1r, bs1)
    sh1L, sh1R, sy1L, sy1R = _make_segsum(2)(h1L, h1R, y1L, y1R, srcs, dsts)
    y2L, y2R, xt_pool = _tc2(sh1L, sh1R, sy1L, sy1R, h1L, h1R, y1L, y1R,
                             deg128, batch2d, Ws2l, Ws2r, bs2, Wg1, bg1)
    sy2L, sy2R = _make_segsum(1)(y2L, y2R, srcs, dsts)
    return _tc3(sy2L, sy2R, y2L, y2R, deg128, batch2d, Wg2, bg2, xt_pool,
                Wat, bat, Waf, baf, Wq, bq, Wl1, bl1, Wl2, bl2)
